# baseline (speedup 1.0000x reference)
import functools

import jax
import jax.numpy as jnp
from jax import lax
from jax.experimental import pallas as pl
from jax.experimental.pallas import tpu as pltpu
from jax.experimental.pallas import tpu_sc as plsc

VOCAB = 1000000
EMBED_DIM = 64
BATCH = 4096
SEQ = 200

NUM_CORES = 2
NUM_SUBCORES = 16
NW = NUM_CORES * NUM_SUBCORES
TOTAL = BATCH * SEQ
PER_W = TOTAL // NW
CHUNK = 128
STEPS = PER_W // CHUNK

_mesh = plsc.VectorSubcoreMesh(core_axis_name="c", subcore_axis_name="s")


@functools.partial(
    pl.kernel,
    out_type=jax.ShapeDtypeStruct((TOTAL, EMBED_DIM), jnp.float32),
    mesh=_mesh,
    scratch_types=[
        pltpu.VMEM((STEPS, CHUNK), jnp.int32),
        pltpu.VMEM((CHUNK, EMBED_DIM), jnp.float32),
        pltpu.SemaphoreType.DMA,
    ],
    compiler_params=pltpu.CompilerParams(use_tc_tiling_on_sc=False),
)
def _sc_gather(idx_hbm, table_hbm, out_hbm, idx_v, rows_v, sem):
    wid = lax.axis_index("s") * NUM_CORES + lax.axis_index("c")
    base = wid * PER_W
    pltpu.sync_copy(idx_hbm.at[wid], idx_v)

    def body(j, carry):
        pltpu.async_copy(table_hbm.at[idx_v.at[j]], rows_v, sem).wait()
        pltpu.sync_copy(rows_v, out_hbm.at[pl.ds(base + j * CHUNK, CHUNK)])
        return carry

    lax.fori_loop(0, STEPS, body, 0)


def kernel(x, embedding_table):
    idx = x.reshape(NW, STEPS, CHUNK).astype(jnp.int32)
    out = _sc_gather(idx, embedding_table)
    return out.reshape(BATCH, SEQ, EMBED_DIM)


# speedup vs baseline: 1.1131x; 1.1131x over previous
import functools

import jax
import jax.numpy as jnp
from jax import lax
from jax.experimental import pallas as pl
from jax.experimental.pallas import tpu as pltpu
from jax.experimental.pallas import tpu_sc as plsc

VOCAB = 1000000
EMBED_DIM = 64
BATCH = 4096
SEQ = 200

NUM_CORES = 2
NUM_SUBCORES = 16
NW = NUM_CORES * NUM_SUBCORES
TOTAL = BATCH * SEQ
PER_W = TOTAL // NW
CHUNK = 128
STEPS = PER_W // CHUNK
K = 4
GROUPS = STEPS // K
NBUF = 2

_mesh = plsc.VectorSubcoreMesh(core_axis_name="c", subcore_axis_name="s")


@functools.partial(
    pl.kernel,
    out_type=jax.ShapeDtypeStruct((TOTAL // CHUNK, CHUNK, EMBED_DIM),
                                  jnp.float32),
    mesh=_mesh,
    scratch_types=[
        pltpu.VMEM((STEPS, CHUNK), jnp.int32),
        pltpu.VMEM((NBUF, K, CHUNK, EMBED_DIM), jnp.float32),
        pltpu.SemaphoreType.DMA,
        pltpu.SemaphoreType.DMA,
    ],
    compiler_params=pltpu.CompilerParams(use_tc_tiling_on_sc=False),
)
def _sc_gather(idx_hbm, table_hbm, out_hbm, idx_v, rows_v, gsem, osem):
    wid = lax.axis_index("s") * NUM_CORES + lax.axis_index("c")
    block0 = wid * STEPS
    pltpu.sync_copy(idx_hbm.at[wid], idx_v)

    def group(g, carry):
        b = lax.rem(g, NBUF)
        @pl.when(g >= NBUF)
        def _():
            pltpu.make_async_copy(
                rows_v.at[b], out_hbm.at[pl.ds(block0, K)], osem).wait()

        cps = [
            pltpu.async_copy(
                table_hbm.at[idx_v.at[g * K + t]], rows_v.at[b, t], gsem)
            for t in range(K)
        ]
        for cp in cps:
            cp.wait()
        pltpu.async_copy(
            rows_v.at[b], out_hbm.at[pl.ds(block0 + g * K, K)], osem)
        return carry

    lax.fori_loop(0, GROUPS, group, 0)
    for _ in range(NBUF):
        pltpu.make_async_copy(
            rows_v.at[0], out_hbm.at[pl.ds(block0, K)], osem).wait()


def kernel(x, embedding_table):
    idx = x.reshape(NW, STEPS, CHUNK).astype(jnp.int32)
    out = _sc_gather(idx, embedding_table)
    return out.reshape(BATCH, SEQ, EMBED_DIM)
